# baseline (speedup 1.0000x reference)
import functools

import jax
import jax.numpy as jnp
from jax import lax
from jax.experimental import pallas as pl
from jax.experimental.pallas import tpu as pltpu
from jax.experimental.pallas import tpu_sc as plsc

QB = 128
QB_C = 32
K_SEL = 30
K_NRM = 8
TD = 16


def _select_body(xw_ref, vpt_ref, idx_ref, key_ref):
    xw = xw_ref[...]
    vpt = vpt_ref[...]
    prod = (xw[:, 0:1] * vpt[0:1, :] + xw[:, 1:2] * vpt[1:2, :]
            + xw[:, 2:3] * vpt[2:3, :])
    pn2 = (vpt[0:1, :] * vpt[0:1, :] + vpt[1:2, :] * vpt[1:2, :]
           + vpt[2:3, :] * vpt[2:3, :])
    key_ref[...] = pn2 - 2.0 * prod
    big = jnp.float32(3.0e38)
    m_tot = vpt.shape[1]
    cl = 128
    ns = m_tot // cl
    kk3 = key_ref[...].reshape(QB, ns, cl)
    cmin = jnp.min(kk3, axis=1)
    iota_t = lax.broadcasted_iota(jnp.int32, (QB, cl), 1).astype(jnp.float32)
    iota_s = lax.broadcasted_iota(jnp.int32, (QB, ns), 1).astype(jnp.float32)
    vlast = jnp.full((QB, 1), -big, jnp.float32)
    tlast = jnp.full((QB, 1), -1.0, jnp.float32)
    slast = jnp.full((QB, 1), -1.0, jnp.float32)
    for k in range(K_SEL):
        m = jnp.min(cmin, axis=1, keepdims=True)
        tsel = jnp.min(jnp.where(cmin == m, iota_t, jnp.float32(cl)),
                       axis=1, keepdims=True)
        onehot = iota_t == tsel
        cv = jnp.sum(jnp.where(onehot[:, None, :], kk3, 0.0), axis=2)
        elig = (cv > vlast) | ((cv == vlast)
                               & ((tsel > tlast)
                                  | ((tsel == tlast) & (iota_s > slast))))
        ssel = jnp.min(jnp.where((cv == m) & elig, iota_s, jnp.float32(ns)),
                       axis=1, keepdims=True)
        idx_ref[:, k : k + 1] = (ssel * cl + tsel).astype(jnp.int32)
        elig2 = (cv > m) | ((cv == m) & (iota_s > ssel))
        newmin = jnp.min(jnp.where(elig2, cv, big), axis=1, keepdims=True)
        cmin = jnp.where(onehot, newmin, cmin)
        vlast, tlast, slast = m, tsel, ssel


def _select_topk(xw, vpt):
    n = xw.shape[0]
    m = vpt.shape[1]
    grid = n // QB
    return pl.pallas_call(
        _select_body,
        grid=(grid,),
        in_specs=[
            pl.BlockSpec((QB, 8), lambda i: (i, 0)),
            pl.BlockSpec((8, m), lambda i: (0, 0)),
        ],
        out_specs=pl.BlockSpec((QB, K_SEL), lambda i: (i, 0)),
        out_shape=jax.ShapeDtypeStruct((n, K_SEL), jnp.int32),
        scratch_shapes=[pltpu.VMEM((QB, m), jnp.float32)],
    )(xw, vpt)


def _sc_gather(table, idx_flat):
    info = plsc.get_sparse_core_info()
    nw = info.num_cores * info.num_subcores
    b = idx_flat.shape[0]
    b_per_w = b // nw
    nchunk = b_per_w // 128
    idx2 = idx_flat.reshape(b // 128, 128)
    mesh = plsc.VectorSubcoreMesh(core_axis_name="c", subcore_axis_name="s")

    @functools.partial(
        pl.kernel,
        mesh=mesh,
        compiler_params=pltpu.CompilerParams(use_tc_tiling_on_sc=False),
        out_type=jax.ShapeDtypeStruct((b // 128, 128, TD), jnp.float32),
        scratch_types=[
            pltpu.VMEM((nchunk, 128), jnp.int32),
            pltpu.VMEM((nchunk, 128, TD), jnp.float32),
            pltpu.SemaphoreType.DMA,
        ],
    )
    def gather_k(table_hbm, idx_hbm, out_hbm, idx_v, rows_v, sem):
        wid = lax.axis_index("s") * info.num_cores + lax.axis_index("c")
        base = wid * nchunk
        pltpu.sync_copy(idx_hbm.at[pl.ds(base, nchunk)], idx_v)

        def chunk(j, _):
            pltpu.async_copy(table_hbm.at[idx_v.at[j]], rows_v.at[j],
                             sem).wait()
            return 0

        lax.fori_loop(0, nchunk, chunk, 0)
        pltpu.sync_copy(rows_v, out_hbm.at[pl.ds(base, nchunk)])

    return gather_k(table, idx2).reshape(b, TD)


def kernel(x_world, voxel_point, voxel_normal, v, W_fc, b_fc, Wq, bq, Wk, bk):
    n = x_world.shape[0]
    m = voxel_point.shape[0]
    xw = x_world.reshape(n, 3)
    xw8 = jnp.pad(xw, ((0, 0), (0, 5)))
    vpt = jnp.pad(voxel_point.T, ((0, 5), (0, 0)))
    idx = _select_topk(xw8, vpt)
    table = jnp.concatenate(
        [voxel_point, voxel_normal, v,
         jnp.zeros((m, TD - 7), jnp.float32)], axis=1)
    gath = _sc_gather(table, idx.reshape(-1))
    g = gath.reshape(n, K_SEL, TD)
    gt = jnp.swapaxes(g, 1, 2)
    wb = jnp.concatenate(
        [W_fc, b_fc[None, :], jnp.zeros((1, W_fc.shape[1]), jnp.float32)],
        axis=0)
    vq = jnp.zeros_like(wb).at[6].set(bq)
    out = _attention(xw, g, gt, wb, vq, Wq, Wk)
    return out


def _attn2_body(xw_ref, g_ref, gt_ref, wb_ref, vq_ref, wq_ref, wkt_ref,
                wbt_ref, out_ref):
    hp = lax.Precision.HIGHEST
    mq = jnp.dot(wb_ref[...], wq_ref[...], precision=hp,
                 preferred_element_type=jnp.float32) + vq_ref[...]
    mkt = jnp.dot(wkt_ref[...], wbt_ref[...], precision=hp,
                  preferred_element_type=jnp.float32)
    gmat = jnp.dot(mq, mkt, precision=hp,
                   preferred_element_type=jnp.float32) * (1.0 / 16.0)

    g = g_ref[...]
    gt = gt_ref[...]
    xw = xw_ref[...]
    pts = g[:, :, 0:3]
    nrm = g[:, :, 3:6]
    xn = jnp.mean(g[:, 0:K_NRM, 3:6], axis=1)
    pos_rel = xw[:, None, :] - pts
    nrm_rel = xn[:, None, :] - nrm
    ones = jnp.ones((QB_C, K_SEL, 1), jnp.float32)
    zeros = jnp.zeros((QB_C, K_SEL, 1), jnp.float32)
    xfeat = jnp.concatenate([pos_rel, nrm_rel, ones, zeros], axis=2)

    pts_t = gt[:, 0:3, :]
    nrm_t = gt[:, 3:6, :]
    vt = gt[:, 6:7, :]
    pos_rel_t = xw[:, :, None] - pts_t
    nrm_rel_t = xn[:, :, None] - nrm_t
    ones_t = jnp.ones((QB_C, 1, K_SEL), jnp.float32)
    zeros_t = jnp.zeros((QB_C, 1, K_SEL), jnp.float32)
    xft = jnp.concatenate([pos_rel_t, nrm_rel_t, ones_t, zeros_t], axis=1)

    y = jnp.dot(xfeat.reshape(QB_C * K_SEL, 8), gmat, precision=hp,
                preferred_element_type=jnp.float32).reshape(QB_C, K_SEL, 8)
    logits = jnp.zeros((QB_C, K_SEL, K_SEL), jnp.float32)
    for c in range(8):
        logits = logits + y[:, :, c : c + 1] * xft[:, c : c + 1, :]

    mx = jnp.max(logits, axis=2, keepdims=True)
    e = jnp.exp(logits - mx)
    den = jnp.sum(e, axis=2)
    num = jnp.sum(e * vt, axis=2)
    res = jnp.mean(num / den, axis=1)
    out_ref[0, 0, :] = res


def _attention(xw, g, gt, wb, vq, Wq, Wk):
    n = xw.shape[0]
    grid = n // QB_C
    cout = Wq.shape[0]
    out3 = pl.pallas_call(
        _attn2_body,
        grid=(grid,),
        in_specs=[
            pl.BlockSpec((QB_C, 3), lambda i: (i, 0)),
            pl.BlockSpec((QB_C, K_SEL, TD), lambda i: (i, 0, 0)),
            pl.BlockSpec((QB_C, TD, K_SEL), lambda i: (i, 0, 0)),
            pl.BlockSpec((8, cout), lambda i: (0, 0)),
            pl.BlockSpec((8, cout), lambda i: (0, 0)),
            pl.BlockSpec((cout, cout), lambda i: (0, 0)),
            pl.BlockSpec((cout, cout), lambda i: (0, 0)),
            pl.BlockSpec((cout, 8), lambda i: (0, 0)),
        ],
        out_specs=pl.BlockSpec((1, 1, QB_C), lambda i: (i, 0, 0)),
        out_shape=jax.ShapeDtypeStruct((grid, 1, QB_C), jnp.float32),
        compiler_params=pltpu.CompilerParams(
            vmem_limit_bytes=100 * 1024 * 1024),
    )(xw, g, gt, wb, vq, Wq, Wk.T, wb.T)
    return out3.reshape(n)


# speedup vs baseline: 4.2250x; 4.2250x over previous
import functools

import jax
import jax.numpy as jnp
from jax import lax
from jax.experimental import pallas as pl
from jax.experimental.pallas import tpu as pltpu
from jax.experimental.pallas import tpu_sc as plsc

QB = 128
QB_C = 32
K_SEL = 30
K_NRM = 8
TD = 16


def _select_body(xw_ref, vpt_ref, idx_ref, key_ref):
    xw = xw_ref[...]
    vpt = vpt_ref[...]
    prod = (xw[:, 0:1] * vpt[0:1, :] + xw[:, 1:2] * vpt[1:2, :]
            + xw[:, 2:3] * vpt[2:3, :])
    pn2 = (vpt[0:1, :] * vpt[0:1, :] + vpt[1:2, :] * vpt[1:2, :]
           + vpt[2:3, :] * vpt[2:3, :])
    key_ref[...] = pn2 - 2.0 * prod
    m_tot = vpt.shape[1]
    fiota = lax.broadcasted_iota(jnp.int32, key_ref.shape, 1).astype(
        jnp.float32)
    big = jnp.float32(3.0e38)
    for k in range(K_SEL):
        kk = key_ref[...]
        m = jnp.min(kk, axis=1, keepdims=True)
        idxf = jnp.min(jnp.where(kk == m, fiota, jnp.float32(m_tot)), axis=1)
        idx_ref[:, k : k + 1] = idxf.astype(jnp.int32)[:, None]
        key_ref[...] = jnp.where(fiota == idxf[:, None], big, kk)


def _select_topk(xw, vpt):
    n = xw.shape[0]
    m = vpt.shape[1]
    grid = n // QB
    return pl.pallas_call(
        _select_body,
        grid=(grid,),
        in_specs=[
            pl.BlockSpec((QB, 8), lambda i: (i, 0)),
            pl.BlockSpec((8, m), lambda i: (0, 0)),
        ],
        out_specs=pl.BlockSpec((QB, K_SEL), lambda i: (i, 0)),
        out_shape=jax.ShapeDtypeStruct((n, K_SEL), jnp.int32),
        scratch_shapes=[pltpu.VMEM((QB, m), jnp.float32)],
    )(xw, vpt)


def _sc_gather(table, idx_flat):
    info = plsc.get_sparse_core_info()
    nw = info.num_cores * info.num_subcores
    b = idx_flat.shape[0]
    b_per_w = b // nw
    nchunk = b_per_w // 128
    idx2 = idx_flat.reshape(b // 128, 128)
    mesh = plsc.VectorSubcoreMesh(core_axis_name="c", subcore_axis_name="s")

    @functools.partial(
        pl.kernel,
        mesh=mesh,
        compiler_params=pltpu.CompilerParams(use_tc_tiling_on_sc=False),
        out_type=jax.ShapeDtypeStruct((b // 128, 128, TD), jnp.float32),
        scratch_types=[
            pltpu.VMEM((nchunk, 128), jnp.int32),
            pltpu.VMEM((nchunk, 128, TD), jnp.float32),
            pltpu.SemaphoreType.DMA,
        ],
    )
    def gather_k(table_hbm, idx_hbm, out_hbm, idx_v, rows_v, sem):
        wid = lax.axis_index("s") * info.num_cores + lax.axis_index("c")
        base = wid * nchunk
        pltpu.sync_copy(idx_hbm.at[pl.ds(base, nchunk)], idx_v)

        def chunk(j, _):
            pltpu.async_copy(table_hbm.at[idx_v.at[j]], rows_v.at[j],
                             sem).wait()
            return 0

        lax.fori_loop(0, nchunk, chunk, 0)
        pltpu.sync_copy(rows_v, out_hbm.at[pl.ds(base, nchunk)])

    return gather_k(table, idx2).reshape(b, TD)


def kernel(x_world, voxel_point, voxel_normal, v, W_fc, b_fc, Wq, bq, Wk, bk):
    n = x_world.shape[0]
    m = voxel_point.shape[0]
    xw = x_world.reshape(n, 3)
    xw8 = jnp.pad(xw, ((0, 0), (0, 5)))
    vpt = jnp.pad(voxel_point.T, ((0, 5), (0, 0)))
    idx = _select_topk(xw8, vpt)
    table = jnp.concatenate(
        [voxel_point, voxel_normal, v,
         jnp.zeros((m, TD - 7), jnp.float32)], axis=1)
    gath = _sc_gather(table, idx.reshape(-1))
    g = gath.reshape(n, K_SEL, TD)
    gt = jnp.swapaxes(g, 1, 2)
    wb = jnp.concatenate(
        [W_fc, b_fc[None, :], jnp.zeros((1, W_fc.shape[1]), jnp.float32)],
        axis=0)
    vq = jnp.zeros_like(wb).at[6].set(bq)
    out = _attention(xw, g, gt, wb, vq, Wq, Wk)
    return out


def _attn2_body(xw_ref, g_ref, gt_ref, wb_ref, vq_ref, wq_ref, wkt_ref,
                wbt_ref, out_ref):
    hp = lax.Precision.HIGHEST
    mq = jnp.dot(wb_ref[...], wq_ref[...], precision=hp,
                 preferred_element_type=jnp.float32) + vq_ref[...]
    mkt = jnp.dot(wkt_ref[...], wbt_ref[...], precision=hp,
                  preferred_element_type=jnp.float32)
    gmat = jnp.dot(mq, mkt, precision=hp,
                   preferred_element_type=jnp.float32) * (1.0 / 16.0)

    g = g_ref[...]
    gt = gt_ref[...]
    xw = xw_ref[...]
    pts = g[:, :, 0:3]
    nrm = g[:, :, 3:6]
    xn = jnp.mean(g[:, 0:K_NRM, 3:6], axis=1)
    pos_rel = xw[:, None, :] - pts
    nrm_rel = xn[:, None, :] - nrm
    ones = jnp.ones((QB_C, K_SEL, 1), jnp.float32)
    zeros = jnp.zeros((QB_C, K_SEL, 1), jnp.float32)
    xfeat = jnp.concatenate([pos_rel, nrm_rel, ones, zeros], axis=2)

    pts_t = gt[:, 0:3, :]
    nrm_t = gt[:, 3:6, :]
    vt = gt[:, 6:7, :]
    pos_rel_t = xw[:, :, None] - pts_t
    nrm_rel_t = xn[:, :, None] - nrm_t
    ones_t = jnp.ones((QB_C, 1, K_SEL), jnp.float32)
    zeros_t = jnp.zeros((QB_C, 1, K_SEL), jnp.float32)
    xft = jnp.concatenate([pos_rel_t, nrm_rel_t, ones_t, zeros_t], axis=1)

    y = jnp.dot(xfeat.reshape(QB_C * K_SEL, 8), gmat, precision=hp,
                preferred_element_type=jnp.float32).reshape(QB_C, K_SEL, 8)
    logits = jnp.zeros((QB_C, K_SEL, K_SEL), jnp.float32)
    for c in range(8):
        logits = logits + y[:, :, c : c + 1] * xft[:, c : c + 1, :]

    mx = jnp.max(logits, axis=2, keepdims=True)
    e = jnp.exp(logits - mx)
    den = jnp.sum(e, axis=2)
    num = jnp.sum(e * vt, axis=2)
    res = jnp.mean(num / den, axis=1)
    out_ref[0, 0, :] = res


def _attention(xw, g, gt, wb, vq, Wq, Wk):
    n = xw.shape[0]
    grid = n // QB_C
    cout = Wq.shape[0]
    out3 = pl.pallas_call(
        _attn2_body,
        grid=(grid,),
        in_specs=[
            pl.BlockSpec((QB_C, 3), lambda i: (i, 0)),
            pl.BlockSpec((QB_C, K_SEL, TD), lambda i: (i, 0, 0)),
            pl.BlockSpec((QB_C, TD, K_SEL), lambda i: (i, 0, 0)),
            pl.BlockSpec((8, cout), lambda i: (0, 0)),
            pl.BlockSpec((8, cout), lambda i: (0, 0)),
            pl.BlockSpec((cout, cout), lambda i: (0, 0)),
            pl.BlockSpec((cout, cout), lambda i: (0, 0)),
            pl.BlockSpec((cout, 8), lambda i: (0, 0)),
        ],
        out_specs=pl.BlockSpec((1, 1, QB_C), lambda i: (i, 0, 0)),
        out_shape=jax.ShapeDtypeStruct((grid, 1, QB_C), jnp.float32),
        compiler_params=pltpu.CompilerParams(
            vmem_limit_bytes=100 * 1024 * 1024),
    )(xw, g, gt, wb, vq, Wq, Wk.T, wb.T)
    return out3.reshape(n)
